# baseline (speedup 1.0000x reference)
import functools
import math

import jax
import jax.numpy as jnp
from jax import lax
from jax.experimental import pallas as pl
from jax.experimental.pallas import tpu as pltpu
from jax.experimental.pallas import tpu_sc as plsc

VOCAB = 100000
EMBED_DIM = 2048
BATCH = 4
SEQ_LEN = 4096

NC, NS, LANES = 2, 16, 16
NW = NC * NS
PW = SEQ_LEN // NW
CHUNK = 16
VECS_PER_ROW = EMBED_DIM // LANES


def _sinusoidal_pe(seq_len: int, d: int) -> jnp.ndarray:
    pos = jnp.arange(seq_len, dtype=jnp.float32)[:, None]
    div = jnp.exp(jnp.arange(0, d, 2, dtype=jnp.float32) * (-math.log(10000.0) / d))
    pe = jnp.zeros((seq_len, d), dtype=jnp.float32)
    pe = pe.at[:, 0::2].set(jnp.sin(pos * div))
    pe = pe.at[:, 1::2].set(jnp.cos(pos * div))
    return pe


def _make_sc_kernel():
    mesh = plsc.VectorSubcoreMesh(
        core_axis_name="c", subcore_axis_name="s",
        num_cores=NC, num_subcores=NS,
    )

    @functools.partial(
        pl.kernel,
        out_type=jax.ShapeDtypeStruct((BATCH, SEQ_LEN, EMBED_DIM), jnp.float32),
        mesh=mesh,
        scratch_types=[
            pltpu.VMEM((CHUNK,), jnp.int32),
            pltpu.VMEM((CHUNK, EMBED_DIM), jnp.float32),
            pltpu.VMEM((CHUNK, EMBED_DIM), jnp.float32),
            pltpu.SemaphoreType.DMA,
        ],
    )
    def body(x_hbm, pe_hbm, table_hbm, out_hbm, idx_v, pe_v, row_v, sem):
        wid = lax.axis_index("s") * NC + lax.axis_index("c")
        pos0 = wid * PW

        def chunk_body(ci, _):
            pos = pos0 + ci * CHUNK
            pltpu.sync_copy(pe_hbm.at[pl.ds(pos, CHUNK)], pe_v)

            def batch_body(b, _):
                pltpu.sync_copy(x_hbm.at[b, pl.ds(pos, CHUNK)], idx_v)
                pltpu.async_copy(table_hbm.at[idx_v], row_v, sem).wait()

                def row_body(r, _):
                    for j in range(VECS_PER_ROW):
                        sl = pl.ds(j * LANES, LANES)
                        plsc.addupdate(row_v.at[r, sl], pe_v[r, sl])
                    return 0

                lax.fori_loop(0, CHUNK, row_body, 0)
                pltpu.sync_copy(row_v, out_hbm.at[b, pl.ds(pos, CHUNK)])
                return 0

            lax.fori_loop(0, BATCH, batch_body, 0)
            return 0

        lax.fori_loop(0, PW // CHUNK, chunk_body, 0)

    return body


_sc_kernel = _make_sc_kernel()


def kernel(x, table):
    pe = _sinusoidal_pe(SEQ_LEN, EMBED_DIM)
    return _sc_kernel(x.astype(jnp.int32), pe, table)
